# baseline (speedup 1.0000x reference)
import functools

import jax
import jax.numpy as jnp
from jax import lax
from jax.experimental import pallas as pl
from jax.experimental.pallas import tpu as pltpu
from jax.experimental.pallas import tpu_sc as plsc

B, N, S, K, D = 8, 2048, 512, 32, 256
NC, NS, L = 2, 16, 16
DI = D // L
CB = 2
NCHUNK = B // CB
CPAIRS = CB * S
PPW = CPAIRS // (NC * NS)
NVALS = float(S * K * D)
_NACC = 4


def _topk_body(nxyz_ref, xyzt_ref, idx_ref):
    b = pl.program_id(0)
    sx = nxyz_ref[0]
    dx = xyzt_ref[0]
    s0 = sx[:, 0:1]
    s1 = sx[:, 1:2]
    s2 = sx[:, 2:3]
    d0 = dx[0:1, :]
    d1 = dx[1:2, :]
    d2 = dx[2:3, :]

    def _r(x):
        return x.astype(jnp.bfloat16).astype(jnp.float32)

    dot = (_r(s0) * _r(d0) + _r(s1) * _r(d1)) + _r(s2) * _r(d2)
    sn = (s0 * s0 + s1 * s1) + s2 * s2
    dn = (d0 * d0 + d1 * d1) + d2 * d2
    dist = (-2.0 * dot + sn) + dn

    lane = lax.broadcasted_iota(jnp.int32, (S, N), 1)
    kcol = lax.broadcasted_iota(jnp.int32, (S, K), 1)
    idxs = jnp.zeros((S, K), jnp.int32)
    inf = jnp.float32(jnp.inf)
    for k in range(K):
        m = jnp.min(dist, axis=1, keepdims=True)
        eq = dist == m
        cand = jnp.where(eq, lane, N)
        a = jnp.min(cand, axis=1, keepdims=True)
        idxs = jnp.where(kcol == k, a, idxs)
        dist = jnp.where(lane == a, inf, dist)
    idx_ref[0] = idxs + b * N


@functools.cache
def _topk_chunk():
    return pl.pallas_call(
        _topk_body,
        grid=(CB,),
        in_specs=[
            pl.BlockSpec((1, S, 3), lambda b: (b, 0, 0)),
            pl.BlockSpec((1, 3, N), lambda b: (b, 0, 0)),
        ],
        out_specs=pl.BlockSpec((1, S, K), lambda b: (b, 0, 0)),
        out_shape=jax.ShapeDtypeStruct((CB, S, K), jnp.int32),
    )


@functools.cache
def _mesh():
    return plsc.VectorSubcoreMesh(core_axis_name="c", subcore_axis_name="s")


def _sc_stats_body(base, points_hbm, np_hbm, idx_hbm, part_hbm,
                   idxslab, npslab, rowsv, partv, gsem0, gsem1):
    cid = lax.axis_index("c")
    sid = lax.axis_index("s")
    lbase = pl.multiple_of(cid * S + sid * PPW, PPW)
    gbase = pl.multiple_of(base + lbase, PPW)

    pltpu.sync_copy(idx_hbm.at[pl.ds(lbase, PPW)], idxslab)
    pltpu.sync_copy(np_hbm.at[pl.ds(gbase, PPW)], npslab)

    def start_gather(j, slot, sem):
        pltpu.async_copy(points_hbm.at[idxslab.at[j]], rowsv.at[slot], sem)

    def wait_gather(j, slot, sem):
        pltpu.make_async_copy(points_hbm.at[idxslab.at[j]],
                              rowsv.at[slot], sem).wait()

    start_gather(0, 0, gsem0)
    start_gather(1, 1, gsem1)

    zero = jnp.zeros((L,), jnp.float32)
    acc_init = tuple([zero] * (2 * _NACC))

    def accum_pair(j, slot, accs):
        accs = list(accs)
        for i in range(DI):
            npi = npslab[j, pl.ds(i * L, L)]
            a1 = accs[i % _NACC]
            a2 = accs[_NACC + i % _NACC]
            for k in range(K):
                r = rowsv[slot, k, pl.ds(i * L, L)] - npi
                a1 = a1 + r
                a2 = a2 + r * r
            accs[i % _NACC] = a1
            accs[_NACC + i % _NACC] = a2
        return tuple(accs)

    def stats_body(jj, accs):
        j0 = 2 * jj
        wait_gather(j0, 0, gsem0)

        @pl.when(jj < PPW // 2 - 1)
        def _():
            start_gather(j0 + 2, 0, gsem0)

        accs = accum_pair(j0, 0, accs)
        wait_gather(j0 + 1, 1, gsem1)

        @pl.when(jj < PPW // 2 - 1)
        def _():
            start_gather(j0 + 3, 1, gsem1)

        return accum_pair(j0 + 1, 1, accs)

    accs = lax.fori_loop(0, PPW // 2, stats_body, acc_init)
    a1 = accs[0]
    a2 = accs[_NACC]
    for i in range(1, _NACC):
        a1 = a1 + accs[i]
        a2 = a2 + accs[_NACC + i]
    partv[pl.ds(0, L)] = a1
    partv[pl.ds(L, L)] = a2
    wrow = pl.multiple_of(cid * NS + sid, 1)
    pltpu.sync_copy(partv, part_hbm.at[wrow])


@functools.cache
def _sc_stats(base):
    return pl.kernel(
        functools.partial(_sc_stats_body, base),
        out_type=jax.ShapeDtypeStruct((NC * NS, 2 * L), jnp.float32),
        mesh=_mesh(),
        scratch_types=[
            pltpu.VMEM((PPW, K), jnp.int32),
            pltpu.VMEM((PPW, D), jnp.float32),
            pltpu.VMEM((2, K, D), jnp.float32),
            pltpu.VMEM((2 * L,), jnp.float32),
            pltpu.SemaphoreType.DMA,
            pltpu.SemaphoreType.DMA,
        ],
    )


def _sc_out_body(base, points_hbm, np_hbm, idx_hbm, sa_hbm, beta_hbm, out_hbm,
                 idxslab, npslab, rowsv, sav, betav, outt,
                 gsem0, gsem1, wsem0, wsem1):
    cid = lax.axis_index("c")
    sid = lax.axis_index("s")
    lbase = pl.multiple_of(cid * S + sid * PPW, PPW)
    gbase = pl.multiple_of(base + lbase, PPW)

    pltpu.sync_copy(idx_hbm.at[pl.ds(lbase, PPW)], idxslab)
    pltpu.sync_copy(np_hbm.at[pl.ds(gbase, PPW)], npslab)
    pltpu.sync_copy(sa_hbm.at[cid], sav)
    pltpu.sync_copy(beta_hbm, betav)

    def start_gather(j, slot, sem):
        pltpu.async_copy(points_hbm.at[idxslab.at[j]], rowsv.at[slot], sem)

    def wait_gather(j, slot, sem):
        pltpu.make_async_copy(points_hbm.at[idxslab.at[j]],
                              rowsv.at[slot], sem).wait()

    def start_write(j, slot, sem):
        row = pl.multiple_of((gbase + j) * K, K)
        pltpu.async_copy(outt.at[slot], out_hbm.at[pl.ds(row, K)], sem)

    def wait_write(j, slot, sem):
        row = pl.multiple_of((gbase + j) * K, K)
        pltpu.make_async_copy(outt.at[slot],
                              out_hbm.at[pl.ds(row, K)], sem).wait()

    start_gather(0, 0, gsem0)
    start_gather(1, 1, gsem1)

    def compute_pair(j, slot):
        for i in range(DI):
            npi = npslab[j, pl.ds(i * L, L)]
            sai = sav[pl.ds(i * L, L)]
            bi = betav[pl.ds(i * L, L)]
            for k in range(K):
                r = rowsv[slot, k, pl.ds(i * L, L)]
                outt[slot, k, pl.ds(i * L, L)] = (r - npi) * sai + bi
                outt[slot, k, pl.ds(D + i * L, L)] = npi

    def half(jj, j, slot, gsem, wsem):
        wait_gather(j, slot, gsem)

        @pl.when(jj > 0)
        def _():
            wait_write(j - 2, slot, wsem)

        compute_pair(j, slot)
        start_write(j, slot, wsem)

        @pl.when(jj < PPW // 2 - 1)
        def _():
            start_gather(j + 2, slot, gsem)

    def out_body(jj, carry):
        j0 = 2 * jj
        half(jj, j0, 0, gsem0, wsem0)
        half(jj, j0 + 1, 1, gsem1, wsem1)
        return carry

    lax.fori_loop(0, PPW // 2, out_body, 0)
    wait_write(PPW - 2, 0, wsem0)
    wait_write(PPW - 1, 1, wsem1)


@functools.cache
def _sc_out(base):
    return pl.kernel(
        functools.partial(_sc_out_body, base),
        out_type=(),
        mesh=_mesh(),
        scratch_types=[
            pltpu.VMEM((PPW, K), jnp.int32),
            pltpu.VMEM((PPW, D), jnp.float32),
            pltpu.VMEM((2, K, D), jnp.float32),
            pltpu.VMEM((D,), jnp.float32),
            pltpu.VMEM((D,), jnp.float32),
            pltpu.VMEM((2, K, 2 * D), jnp.float32),
            pltpu.SemaphoreType.DMA,
            pltpu.SemaphoreType.DMA,
            pltpu.SemaphoreType.DMA,
            pltpu.SemaphoreType.DMA,
        ],
    )


def kernel(xyz, points, new_xyz, new_points, affine_alpha, affine_beta):
    points_flat = points.reshape(B * N, D)
    np_flat = new_points.reshape(B * S, D)
    alpha = affine_alpha.reshape(1, D)
    beta = affine_beta.reshape(D)
    xyzt = jnp.transpose(xyz, (0, 2, 1))

    oref = jax.new_ref(jnp.zeros((B * S * K, 2 * D), jnp.float32))
    topk = _topk_chunk()
    idx_cs = []
    for c in range(NCHUNK):
        b0 = c * CB
        idx_c = topk(new_xyz[b0:b0 + CB], xyzt[b0:b0 + CB])
        idx_cs.append((idx_c + b0 * N).reshape(CPAIRS, K))
    sas = []
    for c in range(NCHUNK):
        b0 = c * CB
        parts = _sc_stats(b0 * S)(points_flat, np_flat, idx_cs[c])
        per_b = parts.reshape(CB, NS, 2, L)
        sums = jnp.sum(per_b[:, :, 0, :], axis=(1, 2))
        sumsqs = jnp.sum(per_b[:, :, 1, :], axis=(1, 2))
        var = (sumsqs - sums * sums / NVALS) / (NVALS - 1.0)
        std = jnp.sqrt(var)
        sas.append(alpha / (std[:, None] + 1e-05))
    for c in range(NCHUNK):
        b0 = c * CB
        _sc_out(b0 * S)(points_flat, np_flat, idx_cs[c], sas[c], beta, oref)
    out = oref[...]
    return (new_xyz, out.reshape(B, S, K, 2 * D))


# speedup vs baseline: 1.0384x; 1.0384x over previous
import functools

import jax
import jax.numpy as jnp
from jax import lax
from jax.experimental import pallas as pl
from jax.experimental.pallas import tpu as pltpu
from jax.experimental.pallas import tpu_sc as plsc

B, N, S, K, D = 8, 2048, 512, 32, 256
NC, NS, L = 2, 16, 16
DI = D // L
CB = 8
NCHUNK = B // CB
CPAIRS = CB * S
PPW = CPAIRS // (NC * NS)
NVALS = float(S * K * D)
_NACC = 4


def _topk_body(nxyz_ref, xyzt_ref, idx_ref):
    b = pl.program_id(0)
    sx = nxyz_ref[0]
    dx = xyzt_ref[0]
    s0 = sx[:, 0:1]
    s1 = sx[:, 1:2]
    s2 = sx[:, 2:3]
    d0 = dx[0:1, :]
    d1 = dx[1:2, :]
    d2 = dx[2:3, :]

    def _r(x):
        return x.astype(jnp.bfloat16).astype(jnp.float32)

    dot = (_r(s0) * _r(d0) + _r(s1) * _r(d1)) + _r(s2) * _r(d2)
    sn = (s0 * s0 + s1 * s1) + s2 * s2
    dn = (d0 * d0 + d1 * d1) + d2 * d2
    dist = (-2.0 * dot + sn) + dn

    lane = lax.broadcasted_iota(jnp.int32, (S, N), 1)
    kcol = lax.broadcasted_iota(jnp.int32, (S, K), 1)
    idxs = jnp.zeros((S, K), jnp.int32)
    inf = jnp.float32(jnp.inf)
    for k in range(K):
        m = jnp.min(dist, axis=1, keepdims=True)
        eq = dist == m
        cand = jnp.where(eq, lane, N)
        a = jnp.min(cand, axis=1, keepdims=True)
        idxs = jnp.where(kcol == k, a, idxs)
        dist = jnp.where(lane == a, inf, dist)
    idx_ref[0] = idxs + b * N


@functools.cache
def _topk_chunk():
    return pl.pallas_call(
        _topk_body,
        grid=(CB,),
        in_specs=[
            pl.BlockSpec((1, S, 3), lambda b: (b, 0, 0)),
            pl.BlockSpec((1, 3, N), lambda b: (b, 0, 0)),
        ],
        out_specs=pl.BlockSpec((1, S, K), lambda b: (b, 0, 0)),
        out_shape=jax.ShapeDtypeStruct((CB, S, K), jnp.int32),
    )


@functools.cache
def _mesh():
    return plsc.VectorSubcoreMesh(core_axis_name="c", subcore_axis_name="s")


def _sc_stats_body(base, points_hbm, np_hbm, idx_hbm, part_hbm,
                   idxslab, npslab, rowsv, partv, gsem0, gsem1):
    cid = lax.axis_index("c")
    sid = lax.axis_index("s")
    lbase = pl.multiple_of((cid * NS + sid) * PPW, PPW)
    gbase = pl.multiple_of(base + lbase, PPW)

    pltpu.sync_copy(idx_hbm.at[pl.ds(lbase, PPW)], idxslab)
    pltpu.sync_copy(np_hbm.at[pl.ds(gbase, PPW)], npslab)

    def start_gather(j, slot, sem):
        pltpu.async_copy(points_hbm.at[idxslab.at[j]], rowsv.at[slot], sem)

    def wait_gather(j, slot, sem):
        pltpu.make_async_copy(points_hbm.at[idxslab.at[j]],
                              rowsv.at[slot], sem).wait()

    start_gather(0, 0, gsem0)
    start_gather(1, 1, gsem1)

    zero = jnp.zeros((L,), jnp.float32)
    acc_init = tuple([zero] * (2 * _NACC))

    def accum_pair(j, slot, accs):
        accs = list(accs)
        for i in range(DI):
            npi = npslab[j, pl.ds(i * L, L)]
            a1 = accs[i % _NACC]
            a2 = accs[_NACC + i % _NACC]
            for k in range(K):
                r = rowsv[slot, k, pl.ds(i * L, L)] - npi
                a1 = a1 + r
                a2 = a2 + r * r
            accs[i % _NACC] = a1
            accs[_NACC + i % _NACC] = a2
        return tuple(accs)

    def stats_body(jj, accs):
        j0 = 2 * jj
        wait_gather(j0, 0, gsem0)

        @pl.when(jj < PPW // 2 - 1)
        def _():
            start_gather(j0 + 2, 0, gsem0)

        accs = accum_pair(j0, 0, accs)
        wait_gather(j0 + 1, 1, gsem1)

        @pl.when(jj < PPW // 2 - 1)
        def _():
            start_gather(j0 + 3, 1, gsem1)

        return accum_pair(j0 + 1, 1, accs)

    accs = lax.fori_loop(0, PPW // 2, stats_body, acc_init)
    a1 = accs[0]
    a2 = accs[_NACC]
    for i in range(1, _NACC):
        a1 = a1 + accs[i]
        a2 = a2 + accs[_NACC + i]
    partv[pl.ds(0, L)] = a1
    partv[pl.ds(L, L)] = a2
    wrow = pl.multiple_of(cid * NS + sid, 1)
    pltpu.sync_copy(partv, part_hbm.at[wrow])


@functools.cache
def _sc_stats(base):
    return pl.kernel(
        functools.partial(_sc_stats_body, base),
        out_type=jax.ShapeDtypeStruct((NC * NS, 2 * L), jnp.float32),
        mesh=_mesh(),
        scratch_types=[
            pltpu.VMEM((PPW, K), jnp.int32),
            pltpu.VMEM((PPW, D), jnp.float32),
            pltpu.VMEM((2, K, D), jnp.float32),
            pltpu.VMEM((2 * L,), jnp.float32),
            pltpu.SemaphoreType.DMA,
            pltpu.SemaphoreType.DMA,
        ],
    )


def _sc_out_body(base, points_hbm, np_hbm, idx_hbm, sa_hbm, beta_hbm, out_hbm,
                 idxslab, npslab, rowsv, sav, betav, outt,
                 gsem0, gsem1, wsem0, wsem1):
    cid = lax.axis_index("c")
    sid = lax.axis_index("s")
    lbase = pl.multiple_of((cid * NS + sid) * PPW, PPW)
    gbase = pl.multiple_of(base + lbase, PPW)
    bid = lbase // S

    pltpu.sync_copy(idx_hbm.at[pl.ds(lbase, PPW)], idxslab)
    pltpu.sync_copy(np_hbm.at[pl.ds(gbase, PPW)], npslab)
    pltpu.sync_copy(sa_hbm.at[bid], sav)
    pltpu.sync_copy(beta_hbm, betav)

    def start_gather(j, slot, sem):
        pltpu.async_copy(points_hbm.at[idxslab.at[j]], rowsv.at[slot], sem)

    def wait_gather(j, slot, sem):
        pltpu.make_async_copy(points_hbm.at[idxslab.at[j]],
                              rowsv.at[slot], sem).wait()

    def start_write(j, slot, sem):
        row = pl.multiple_of((gbase + j) * K, K)
        pltpu.async_copy(outt.at[slot], out_hbm.at[pl.ds(row, K)], sem)

    def wait_write(j, slot, sem):
        row = pl.multiple_of((gbase + j) * K, K)
        pltpu.make_async_copy(outt.at[slot],
                              out_hbm.at[pl.ds(row, K)], sem).wait()

    start_gather(0, 0, gsem0)
    start_gather(1, 1, gsem1)

    def compute_pair(j, slot):
        for i in range(DI):
            npi = npslab[j, pl.ds(i * L, L)]
            sai = sav[pl.ds(i * L, L)]
            bi = betav[pl.ds(i * L, L)]
            for k in range(K):
                r = rowsv[slot, k, pl.ds(i * L, L)]
                outt[slot, k, pl.ds(i * L, L)] = (r - npi) * sai + bi
                outt[slot, k, pl.ds(D + i * L, L)] = npi

    def half(jj, j, slot, gsem, wsem):
        wait_gather(j, slot, gsem)

        @pl.when(jj > 0)
        def _():
            wait_write(j - 2, slot, wsem)

        compute_pair(j, slot)
        start_write(j, slot, wsem)

        @pl.when(jj < PPW // 2 - 1)
        def _():
            start_gather(j + 2, slot, gsem)

    def out_body(jj, carry):
        j0 = 2 * jj
        half(jj, j0, 0, gsem0, wsem0)
        half(jj, j0 + 1, 1, gsem1, wsem1)
        return carry

    lax.fori_loop(0, PPW // 2, out_body, 0)
    wait_write(PPW - 2, 0, wsem0)
    wait_write(PPW - 1, 1, wsem1)


@functools.cache
def _sc_out(base):
    return pl.kernel(
        functools.partial(_sc_out_body, base),
        out_type=(),
        mesh=_mesh(),
        scratch_types=[
            pltpu.VMEM((PPW, K), jnp.int32),
            pltpu.VMEM((PPW, D), jnp.float32),
            pltpu.VMEM((2, K, D), jnp.float32),
            pltpu.VMEM((D,), jnp.float32),
            pltpu.VMEM((D,), jnp.float32),
            pltpu.VMEM((2, K, 2 * D), jnp.float32),
            pltpu.SemaphoreType.DMA,
            pltpu.SemaphoreType.DMA,
            pltpu.SemaphoreType.DMA,
            pltpu.SemaphoreType.DMA,
        ],
    )


def kernel(xyz, points, new_xyz, new_points, affine_alpha, affine_beta):
    points_flat = points.reshape(B * N, D)
    np_flat = new_points.reshape(B * S, D)
    alpha = affine_alpha.reshape(1, D)
    beta = affine_beta.reshape(D)
    xyzt = jnp.transpose(xyz, (0, 2, 1))

    oref = jax.new_ref(jnp.zeros((B * S * K, 2 * D), jnp.float32))
    topk = _topk_chunk()
    idx_cs = []
    for c in range(NCHUNK):
        b0 = c * CB
        idx_c = topk(new_xyz[b0:b0 + CB], xyzt[b0:b0 + CB])
        idx_cs.append((idx_c + b0 * N).reshape(CPAIRS, K))
    sas = []
    for c in range(NCHUNK):
        b0 = c * CB
        parts = _sc_stats(b0 * S)(points_flat, np_flat, idx_cs[c])
        per_b = parts.reshape(CB, (NC * NS) // CB, 2, L)
        sums = jnp.sum(per_b[:, :, 0, :], axis=(1, 2))
        sumsqs = jnp.sum(per_b[:, :, 1, :], axis=(1, 2))
        var = (sumsqs - sums * sums / NVALS) / (NVALS - 1.0)
        std = jnp.sqrt(var)
        sas.append(alpha / (std[:, None] + 1e-05))
    for c in range(NCHUNK):
        b0 = c * CB
        _sc_out(b0 * S)(points_flat, np_flat, idx_cs[c], sas[c], beta, oref)
    out = oref[...]
    return (new_xyz, out.reshape(B, S, K, 2 * D))


# speedup vs baseline: 1.1531x; 1.1104x over previous
import functools

import jax
import jax.numpy as jnp
from jax import lax
from jax.experimental import pallas as pl
from jax.experimental.pallas import tpu as pltpu
from jax.experimental.pallas import tpu_sc as plsc

B, N, S, K, D = 8, 2048, 512, 32, 256
NC, NS, L = 2, 16, 16
DI = D // L
CB = 8
NCHUNK = B // CB
CPAIRS = CB * S
PPW = CPAIRS // (NC * NS)
NVALS = float(S * K * D)
_NACC = 4


def _topk_body(nxyz_ref, xyzt_ref, idx_ref):
    b = pl.program_id(0)
    sx = nxyz_ref[0]
    dx = xyzt_ref[0]
    s0 = sx[:, 0:1]
    s1 = sx[:, 1:2]
    s2 = sx[:, 2:3]
    d0 = dx[0:1, :]
    d1 = dx[1:2, :]
    d2 = dx[2:3, :]

    def _r(x):
        return x.astype(jnp.bfloat16).astype(jnp.float32)

    dot = (_r(s0) * _r(d0) + _r(s1) * _r(d1)) + _r(s2) * _r(d2)
    sn = (s0 * s0 + s1 * s1) + s2 * s2
    dn = (d0 * d0 + d1 * d1) + d2 * d2
    dist = (-2.0 * dot + sn) + dn

    lane = lax.broadcasted_iota(jnp.int32, (S, N), 1)
    kcol = lax.broadcasted_iota(jnp.int32, (S, K), 1)
    idxs = jnp.zeros((S, K), jnp.int32)
    inf = jnp.float32(jnp.inf)
    for k in range(K):
        m = jnp.min(dist, axis=1, keepdims=True)
        eq = dist == m
        cand = jnp.where(eq, lane, N)
        a = jnp.min(cand, axis=1, keepdims=True)
        idxs = jnp.where(kcol == k, a, idxs)
        dist = jnp.where(lane == a, inf, dist)
    idx_ref[0] = idxs + b * N


@functools.cache
def _topk_chunk():
    return pl.pallas_call(
        _topk_body,
        grid=(CB,),
        in_specs=[
            pl.BlockSpec((1, S, 3), lambda b: (b, 0, 0)),
            pl.BlockSpec((1, 3, N), lambda b: (b, 0, 0)),
        ],
        out_specs=pl.BlockSpec((1, S, K), lambda b: (b, 0, 0)),
        out_shape=jax.ShapeDtypeStruct((CB, S, K), jnp.int32),
    )


@functools.cache
def _mesh():
    return plsc.VectorSubcoreMesh(core_axis_name="c", subcore_axis_name="s")


def _sc_stats_body(base, points_hbm, np_hbm, idx_hbm, part_hbm,
                   idxslab, npslab, rowsv, partv, gsem0, gsem1):
    cid = lax.axis_index("c")
    sid = lax.axis_index("s")
    lbase = pl.multiple_of((cid * NS + sid) * PPW, PPW)
    gbase = pl.multiple_of(base + lbase, PPW)

    pltpu.sync_copy(idx_hbm.at[pl.ds(lbase, PPW)], idxslab)
    pltpu.sync_copy(np_hbm.at[pl.ds(gbase, PPW)], npslab)

    def start_gather(j, slot, sem):
        pltpu.async_copy(points_hbm.at[idxslab.at[j]], rowsv.at[slot], sem)

    def wait_gather(j, slot, sem):
        pltpu.make_async_copy(points_hbm.at[idxslab.at[j]],
                              rowsv.at[slot], sem).wait()

    start_gather(0, 0, gsem0)
    start_gather(1, 1, gsem1)

    zero = jnp.zeros((L,), jnp.float32)
    acc_init = tuple([zero] * (2 * _NACC))

    def accum_pair(j, slot, accs):
        accs = list(accs)
        for i in range(DI):
            npi = npslab[j, pl.ds(i * L, L)]
            a1 = accs[i % _NACC]
            a2 = accs[_NACC + i % _NACC]
            for k in range(K):
                r = rowsv[slot, k, pl.ds(i * L, L)] - npi
                a1 = a1 + r
                a2 = a2 + r * r
            accs[i % _NACC] = a1
            accs[_NACC + i % _NACC] = a2
        return tuple(accs)

    def stats_body(jj, accs):
        j0 = 2 * jj
        wait_gather(j0, 0, gsem0)

        @pl.when(jj < PPW // 2 - 1)
        def _():
            start_gather(j0 + 2, 0, gsem0)

        accs = accum_pair(j0, 0, accs)
        wait_gather(j0 + 1, 1, gsem1)

        @pl.when(jj < PPW // 2 - 1)
        def _():
            start_gather(j0 + 3, 1, gsem1)

        return accum_pair(j0 + 1, 1, accs)

    accs = lax.fori_loop(0, PPW // 2, stats_body, acc_init)
    a1 = accs[0]
    a2 = accs[_NACC]
    for i in range(1, _NACC):
        a1 = a1 + accs[i]
        a2 = a2 + accs[_NACC + i]
    partv[pl.ds(0, L)] = a1
    partv[pl.ds(L, L)] = a2
    wrow = pl.multiple_of(cid * NS + sid, 1)
    pltpu.sync_copy(partv, part_hbm.at[wrow])


@functools.cache
def _sc_stats(base):
    return pl.kernel(
        functools.partial(_sc_stats_body, base),
        out_type=jax.ShapeDtypeStruct((NC * NS, 2 * L), jnp.float32),
        mesh=_mesh(),
        scratch_types=[
            pltpu.VMEM((PPW, K), jnp.int32),
            pltpu.VMEM((PPW, D), jnp.float32),
            pltpu.VMEM((2, K, D), jnp.float32),
            pltpu.VMEM((2 * L,), jnp.float32),
            pltpu.SemaphoreType.DMA,
            pltpu.SemaphoreType.DMA,
        ],
    )


def _sc_out_body(base, points_hbm, np_hbm, idx_hbm, sa_hbm, beta_hbm, out_hbm,
                 idxslab, npslab, rowsv, sav, betav, outt,
                 gsem0, gsem1, wsem0, wsem1):
    cid = lax.axis_index("c")
    sid = lax.axis_index("s")
    lbase = pl.multiple_of((cid * NS + sid) * PPW, PPW)
    gbase = pl.multiple_of(base + lbase, PPW)
    bid = lbase // S

    pltpu.sync_copy(idx_hbm.at[pl.ds(lbase, PPW)], idxslab)
    pltpu.sync_copy(np_hbm.at[pl.ds(gbase, PPW)], npslab)
    pltpu.sync_copy(sa_hbm.at[bid], sav)
    pltpu.sync_copy(beta_hbm, betav)

    def start_gather(j, slot, sem):
        pltpu.async_copy(points_hbm.at[idxslab.at[j]], rowsv.at[slot], sem)

    def wait_gather(j, slot, sem):
        pltpu.make_async_copy(points_hbm.at[idxslab.at[j]],
                              rowsv.at[slot], sem).wait()

    def start_write(j, slot, sem):
        row = pl.multiple_of((gbase + j) * K, K)
        pltpu.async_copy(outt.at[slot], out_hbm.at[pl.ds(row, K)], sem)

    def wait_write(j, slot, sem):
        row = pl.multiple_of((gbase + j) * K, K)
        pltpu.make_async_copy(outt.at[slot],
                              out_hbm.at[pl.ds(row, K)], sem).wait()

    start_gather(0, 0, gsem0)
    start_gather(1, 1, gsem1)

    def compute_pair(j, slot):
        for i in range(DI):
            npi = npslab[j, pl.ds(i * L, L)]
            sai = sav[pl.ds(i * L, L)]
            bi = betav[pl.ds(i * L, L)]
            for k in range(K):
                r = rowsv[slot, k, pl.ds(i * L, L)]
                outt[slot, k, pl.ds(i * L, L)] = (r - npi) * sai + bi
                outt[slot, k, pl.ds(D + i * L, L)] = npi

    def half(jj, j, slot, gsem, wsem):
        wait_gather(j, slot, gsem)

        @pl.when(jj > 0)
        def _():
            wait_write(j - 2, slot, wsem)

        compute_pair(j, slot)
        start_write(j, slot, wsem)

        @pl.when(jj < PPW // 2 - 1)
        def _():
            start_gather(j + 2, slot, gsem)

    def out_body(jj, carry):
        j0 = 2 * jj
        half(jj, j0, 0, gsem0, wsem0)
        half(jj, j0 + 1, 1, gsem1, wsem1)
        return carry

    lax.fori_loop(0, PPW // 2, out_body, 0)
    wait_write(PPW - 2, 0, wsem0)
    wait_write(PPW - 1, 1, wsem1)


@functools.cache
def _sc_out(base):
    return pl.kernel(
        functools.partial(_sc_out_body, base),
        out_type=jax.ShapeDtypeStruct((B * S * K, 2 * D), jnp.float32),
        mesh=_mesh(),
        scratch_types=[
            pltpu.VMEM((PPW, K), jnp.int32),
            pltpu.VMEM((PPW, D), jnp.float32),
            pltpu.VMEM((2, K, D), jnp.float32),
            pltpu.VMEM((D,), jnp.float32),
            pltpu.VMEM((D,), jnp.float32),
            pltpu.VMEM((2, K, 2 * D), jnp.float32),
            pltpu.SemaphoreType.DMA,
            pltpu.SemaphoreType.DMA,
            pltpu.SemaphoreType.DMA,
            pltpu.SemaphoreType.DMA,
        ],
    )


def kernel(xyz, points, new_xyz, new_points, affine_alpha, affine_beta):
    points_flat = points.reshape(B * N, D)
    np_flat = new_points.reshape(B * S, D)
    alpha = affine_alpha.reshape(1, D)
    beta = affine_beta.reshape(D)
    xyzt = jnp.transpose(xyz, (0, 2, 1))

    topk = _topk_chunk()
    idx_cs = []
    for c in range(NCHUNK):
        b0 = c * CB
        idx_c = topk(new_xyz[b0:b0 + CB], xyzt[b0:b0 + CB])
        idx_cs.append((idx_c + b0 * N).reshape(CPAIRS, K))
    sas = []
    for c in range(NCHUNK):
        b0 = c * CB
        parts = _sc_stats(b0 * S)(points_flat, np_flat, idx_cs[c])
        per_b = parts.reshape(CB, (NC * NS) // CB, 2, L)
        sums = jnp.sum(per_b[:, :, 0, :], axis=(1, 2))
        sumsqs = jnp.sum(per_b[:, :, 1, :], axis=(1, 2))
        var = (sumsqs - sums * sums / NVALS) / (NVALS - 1.0)
        std = jnp.sqrt(var)
        sas.append(alpha / (std[:, None] + 1e-05))
    out = _sc_out(0)(points_flat, np_flat, idx_cs[0], sas[0], beta)
    return (new_xyz, out.reshape(B, S, K, 2 * D))
